# baseline (speedup 1.0000x reference)
import functools

import jax
import jax.numpy as jnp
from jax import lax
from jax.experimental import pallas as pl
from jax.experimental.pallas import tpu as pltpu
from jax.experimental.pallas import tpu_sc as plsc

E = 64
K = 8
COEF = 0.01
TB = 1024
NCHUNK = 4
CB = 8
CT = CB * TB
NC, NS, L = 2, 16, 16
NW = NC * NS
TPW = CT // NW
T_TOTAL = NCHUNK * CT
MASK6 = ~63
SENT = -2147483648
AUXN = T_TOTAL * K // NW


def _tc_body(x_ref, w_ref, lt_ref, ps_ref, acc_ref):
    i = pl.program_id(0)
    n = pl.num_programs(0)
    x = x_ref[...]
    w = w_ref[...]
    lt = lax.dot_general(w, x, (((1,), (1,)), ((), ())),
                         preferred_element_type=jnp.float32)

    @pl.when(i == 0)
    def _init():
        acc_ref[...] = jnp.zeros_like(acc_ref)

    colmax = jnp.max(lt, axis=0, keepdims=True)
    ex = jnp.exp(lt - colmax)
    recip = 1.0 / jnp.sum(ex, axis=0, keepdims=True)
    acc_ref[...] = acc_ref[...] + lax.dot_general(
        recip, ex, (((1,), (1,)), ((), ())),
        preferred_element_type=jnp.float32)
    lt_ref[...] = lt[None]

    @pl.when(i == n - 1)
    def _fin():
        ps_ref[...] = acc_ref[...]


def _ord(u):
    return u ^ (lax.shift_right_arithmetic(u, 31) & 0x7FFFFFFF)


def _sc_route_body(lt_hbm, tw_hbm, ti_hbm, lt_v, tw_v, ti_v):
    c = lax.axis_index("c")
    sx = lax.axis_index("s")
    wid = sx * NC + c
    b = wid // 4
    q = wid % 4

    pltpu.sync_copy(lt_hbm.at[b, :, pl.ds(q * TPW, TPW)], lt_v)

    def group(g, carry):
        base = g * L
        t = [jnp.full((L,), SENT, jnp.int32) for _ in range(K)]
        for e in range(E):
            v = lt_v[e, pl.ds(base, L)]
            u = lax.bitcast_convert_type(v, jnp.int32)
            cur = (_ord(u) & MASK6) | (63 - e)
            for j in range(K):
                hi = lax.max(t[j], cur)
                cur = lax.min(t[j], cur)
                t[j] = hi
        exs = []
        idxs = []
        v0 = None
        for j in range(K):
            aj = 63 - (t[j] & 63)
            vj = lax.bitcast_convert_type(_ord(t[j] & MASK6), jnp.float32)
            if j == 0:
                v0 = vj
            idxs.append(aj)
            exs.append(jnp.exp(vj - v0))
        tot = exs[0]
        for j in range(1, K):
            tot = tot + exs[j]
        inv = 1.0 / tot
        for j in range(K):
            tw_v[pl.ds(j * TPW + base, L)] = exs[j] * inv
            ti_v[pl.ds(j * TPW + base, L)] = idxs[j]
        return carry

    lax.fori_loop(0, TPW // L, group, jnp.int32(0))

    for j in range(K):
        pltpu.sync_copy(tw_v.at[pl.ds(j * TPW, TPW)],
                        tw_hbm.at[pl.ds(j * CT + wid * TPW, TPW)])
        pltpu.sync_copy(ti_v.at[pl.ds(j * TPW, TPW)],
                        ti_hbm.at[pl.ds(j * CT + wid * TPW, TPW)])


def _sc_aux_body(ti0, ti1, ti2, ti3, ps0, ps1, ps2, ps3, auxp_hbm,
                 tiv, sv, aux_v):
    c = lax.axis_index("c")
    sx = lax.axis_index("s")
    wid = sx * NC + c
    ch = wid // 8
    off = (wid % 8) * AUXN

    for i, ti_h in enumerate((ti0, ti1, ti2, ti3)):
        @pl.when(ch == i)
        def _(ti_h=ti_h):
            pltpu.sync_copy(ti_h.at[pl.ds(off, AUXN)], tiv)

    for i, ps_h in enumerate((ps0, ps1, ps2, ps3)):
        pltpu.sync_copy(ps_h.at[0], sv.at[pl.ds(i * E, E)])
    scale = COEF * E / (T_TOTAL * T_TOTAL)
    s_tab = []
    for p in range(4):
        tab = sv[pl.ds(p * L, L)]
        for i in range(1, 4):
            tab = tab + sv[pl.ds(i * E + p * L, L)]
        s_tab.append(tab * scale)

    def step(i, acc):
        iv = tiv[pl.ds(i * L, L)]
        p = lax.shift_right_logical(iv, 4)
        wi = iv & 15
        gv = jnp.take(s_tab[3], wi, mode="fill")
        for q in range(2, -1, -1):
            gv = jnp.where(p == q, jnp.take(s_tab[q], wi, mode="fill"), gv)
        return acc + gv

    acc = lax.fori_loop(0, AUXN // L, step, jnp.zeros((L,), jnp.float32))
    aux_v[...] = acc
    pltpu.sync_copy(aux_v, auxp_hbm.at[wid])


@jax.jit
def kernel(hidden_states, gate_w):
    t, h = hidden_states.shape
    mesh = plsc.VectorSubcoreMesh(core_axis_name="c", subcore_axis_name="s")
    scp = pltpu.CompilerParams(use_tc_tiling_on_sc=False)

    sc_route = functools.partial(
        pl.kernel,
        mesh=mesh,
        compiler_params=scp,
        out_type=[jax.ShapeDtypeStruct((K * CT,), jnp.float32),
                  jax.ShapeDtypeStruct((K * CT,), jnp.int32)],
        scratch_types=[pltpu.VMEM((E, TPW), jnp.float32),
                       pltpu.VMEM((K * TPW,), jnp.float32),
                       pltpu.VMEM((K * TPW,), jnp.int32)],
    )(_sc_route_body)

    sc_aux = functools.partial(
        pl.kernel,
        mesh=mesh,
        compiler_params=scp,
        out_type=[jax.ShapeDtypeStruct((NW, L), jnp.float32)],
        scratch_types=[pltpu.VMEM((AUXN,), jnp.int32),
                       pltpu.VMEM((4 * E,), jnp.float32),
                       pltpu.VMEM((L,), jnp.float32)],
    )(_sc_aux_body)

    tws, tis, pss = [], [], []
    for ci in range(NCHUNK):
        lt3, ps = pl.pallas_call(
            _tc_body,
            grid=(CB,),
            in_specs=[pl.BlockSpec((TB, h), lambda i, ci=ci: (ci * CB + i, 0)),
                      pl.BlockSpec((E, h), lambda i: (0, 0))],
            out_specs=[pl.BlockSpec((1, E, TB), lambda i: (i, 0, 0)),
                       pl.BlockSpec((1, E), lambda i: (0, 0))],
            out_shape=[jax.ShapeDtypeStruct((CB, E, TB), jnp.float32),
                       jax.ShapeDtypeStruct((1, E), jnp.float32)],
            scratch_shapes=[pltpu.VMEM((1, E), jnp.float32)],
        )(hidden_states, gate_w)
        twf, tif = sc_route(lt3)
        tws.append(twf.reshape(K, CT))
        tis.append(tif.reshape(K, CT))
        pss.append(ps)

    auxp, = sc_aux(tis[0].reshape(K * CT), tis[1].reshape(K * CT),
                   tis[2].reshape(K * CT), tis[3].reshape(K * CT),
                   pss[0], pss[1], pss[2], pss[3])

    tw = jnp.concatenate(tws, axis=1).T
    ti = jnp.concatenate(tis, axis=1).T
    return tw, ti, jnp.sum(auxp)


# speedup vs baseline: 1.0891x; 1.0891x over previous
import functools

import jax
import jax.numpy as jnp
from jax import lax
from jax.experimental import pallas as pl
from jax.experimental.pallas import tpu as pltpu
from jax.experimental.pallas import tpu_sc as plsc

E = 64
K = 8
COEF = 0.01
TB = 1024
CHUNKS = (16, 8, 8)
NBLK = sum(CHUNKS)
NC, NS, L = 2, 16, 16
NW = NC * NS
T_TOTAL = NBLK * TB
MASK6 = ~63
SENT = -2147483648
AUXN = TB * K


def _tc_body(x_ref, w_ref, lt_ref, ps_ref, acc_ref):
    i = pl.program_id(0)
    n = pl.num_programs(0)
    x = x_ref[...]
    w = w_ref[...]
    lt = lax.dot_general(w, x, (((1,), (1,)), ((), ())),
                         preferred_element_type=jnp.float32)

    @pl.when(i == 0)
    def _init():
        acc_ref[...] = jnp.zeros_like(acc_ref)

    colmax = jnp.max(lt, axis=0, keepdims=True)
    ex = jnp.exp(lt - colmax)
    recip = 1.0 / jnp.sum(ex, axis=0, keepdims=True)
    acc_ref[...] = acc_ref[...] + lax.dot_general(
        recip, ex, (((1,), (1,)), ((), ())),
        preferred_element_type=jnp.float32)
    lt_ref[...] = lt[None]

    @pl.when(i == n - 1)
    def _fin():
        ps_ref[...] = jnp.concatenate(
            [acc_ref[...], jnp.zeros((1, 128 - E), jnp.float32)], axis=1)


def _ord(u):
    return u ^ (lax.shift_right_arithmetic(u, 31) & 0x7FFFFFFF)


def _make_route_body(cb):
    ct = cb * TB
    tpw = ct // NW
    per_blk = TB // tpw

    def _sc_route_body(lt_hbm, tw_hbm, ti_hbm, lt_v, tw_v, ti_v):
        c = lax.axis_index("c")
        sx = lax.axis_index("s")
        wid = sx * NC + c
        b = wid // per_blk
        q = wid % per_blk

        pltpu.sync_copy(lt_hbm.at[b, :, pl.ds(q * tpw, tpw)], lt_v)

        def group(g, carry):
            base = g * L
            t = [jnp.full((L,), SENT, jnp.int32) for _ in range(K)]
            for e in range(E):
                v = lt_v[e, pl.ds(base, L)]
                u = lax.bitcast_convert_type(v, jnp.int32)
                cur = (_ord(u) & MASK6) | (63 - e)
                for j in range(K):
                    hi = lax.max(t[j], cur)
                    cur = lax.min(t[j], cur)
                    t[j] = hi
            exs = []
            idxs = []
            v0 = None
            for j in range(K):
                aj = 63 - (t[j] & 63)
                vj = lax.bitcast_convert_type(_ord(t[j] & MASK6), jnp.float32)
                if j == 0:
                    v0 = vj
                idxs.append(aj)
                exs.append(jnp.exp(vj - v0))
            tot = exs[0]
            for j in range(1, K):
                tot = tot + exs[j]
            inv = 1.0 / tot
            for j in range(K):
                tw_v[pl.ds(j * tpw + base, L)] = exs[j] * inv
                ti_v[pl.ds(j * tpw + base, L)] = idxs[j]
            return carry

        lax.fori_loop(0, tpw // L, group, jnp.int32(0))

        for j in range(K):
            pltpu.sync_copy(tw_v.at[pl.ds(j * tpw, tpw)],
                            tw_hbm.at[pl.ds(j * ct + wid * tpw, tpw)])
            pltpu.sync_copy(ti_v.at[pl.ds(j * tpw, tpw)],
                            ti_hbm.at[pl.ds(j * ct + wid * tpw, tpw)])

    return _sc_route_body, ct, tpw


def _sc_aux_body(ti0, ti1, ti2, ps0, ps1, ps2, auxp_hbm, tiv, sv, aux_v):
    c = lax.axis_index("c")
    sx = lax.axis_index("s")
    wid = sx * NC + c
    starts = []
    acc0 = 0
    for cb in CHUNKS:
        starts.append(acc0)
        acc0 += cb

    for i, ti_h in enumerate((ti0, ti1, ti2)):
        @pl.when(jnp.logical_and(wid >= starts[i],
                                 wid < starts[i] + CHUNKS[i]))
        def _(ti_h=ti_h, st=starts[i]):
            pltpu.sync_copy(ti_h.at[pl.ds((wid - st) * AUXN, AUXN)], tiv)

    for i, ps_h in enumerate((ps0, ps1, ps2)):
        pltpu.sync_copy(ps_h.at[0], sv.at[pl.ds(i * 128, 128)])
    scale = COEF * E / (float(T_TOTAL) * float(T_TOTAL))
    s_tab = []
    for p in range(4):
        tab = sv[pl.ds(p * L, L)]
        for i in range(1, len(CHUNKS)):
            tab = tab + sv[pl.ds(i * 128 + p * L, L)]
        s_tab.append(tab * scale)

    def step(i, acc):
        iv = tiv[pl.ds(i * L, L)]
        p = lax.shift_right_logical(iv, 4)
        wi = iv & 15
        gv = jnp.take(s_tab[3], wi, mode="fill")
        for q in range(2, -1, -1):
            gv = jnp.where(p == q, jnp.take(s_tab[q], wi, mode="fill"), gv)
        return acc + gv

    acc = lax.fori_loop(0, AUXN // L, step, jnp.zeros((L,), jnp.float32))
    aux_v[...] = acc
    pltpu.sync_copy(aux_v, auxp_hbm.at[wid])


@jax.jit
def kernel(hidden_states, gate_w):
    t, h = hidden_states.shape
    mesh = plsc.VectorSubcoreMesh(core_axis_name="c", subcore_axis_name="s")
    scp = pltpu.CompilerParams(use_tc_tiling_on_sc=True)

    tws, tis, pss = [], [], []
    blk0 = 0
    for cb in CHUNKS:
        body, ct, tpw = _make_route_body(cb)
        sc_route = functools.partial(
            pl.kernel,
            mesh=mesh,
            compiler_params=scp,
            out_type=[jax.ShapeDtypeStruct((K * ct,), jnp.float32),
                      jax.ShapeDtypeStruct((K * ct,), jnp.int32)],
            scratch_types=[pltpu.VMEM((E, tpw), jnp.float32),
                           pltpu.VMEM((K * tpw,), jnp.float32),
                           pltpu.VMEM((K * tpw,), jnp.int32)],
        )(body)

        lt3, ps = pl.pallas_call(
            _tc_body,
            grid=(cb,),
            in_specs=[pl.BlockSpec((TB, h), lambda i, b0=blk0: (b0 + i, 0)),
                      pl.BlockSpec((E, h), lambda i: (0, 0))],
            out_specs=[pl.BlockSpec((1, E, TB), lambda i: (i, 0, 0)),
                       pl.BlockSpec((1, 128), lambda i: (0, 0))],
            out_shape=[jax.ShapeDtypeStruct((cb, E, TB), jnp.float32),
                       jax.ShapeDtypeStruct((1, 128), jnp.float32)],
            scratch_shapes=[pltpu.VMEM((1, E), jnp.float32)],
        )(hidden_states, gate_w)
        twf, tif = sc_route(lt3)
        tws.append(twf.reshape(K, ct))
        tis.append(tif.reshape(K, ct))
        pss.append(ps)
        blk0 += cb

    sc_aux = functools.partial(
        pl.kernel,
        mesh=mesh,
        compiler_params=scp,
        out_type=[jax.ShapeDtypeStruct((NW, L), jnp.float32)],
        scratch_types=[pltpu.VMEM((AUXN,), jnp.int32),
                       pltpu.VMEM((len(CHUNKS) * 128,), jnp.float32),
                       pltpu.VMEM((L,), jnp.float32)],
    )(_sc_aux_body)

    auxp, = sc_aux(tis[0].reshape(-1), tis[1].reshape(-1), tis[2].reshape(-1),
                   pss[0], pss[1], pss[2])

    tw = jnp.concatenate(tws, axis=1).T
    ti = jnp.concatenate(tis, axis=1).T
    return tw, ti, jnp.sum(auxp)
